# baseline (speedup 1.0000x reference)
import jax
import jax.numpy as jnp
from jax import lax
from jax.experimental import pallas as pl
from jax.experimental.pallas import tpu as pltpu
from jax.experimental.pallas import tpu_sc as plsc

S = 30.0
M = 0.4

_B = 1024
_C = 100000
_B_TC = 768
_B_SC = _B - _B_TC

_BLOCK_B = 256
_BLOCK_C = 8192

_NW = 32
_ROWS_PW = _B_SC // _NW
_CHUNK = 10000
_T = (_ROWS_PW * _C) // _CHUNK
_NBUF = 4
_VREGS = _CHUNK // 16


def _mcp_block(cosine_ref, label_ref, out_ref):
    j = pl.program_id(1)
    cols = jax.lax.broadcasted_iota(jnp.int32, cosine_ref.shape, 1) + j * _BLOCK_C
    mask = cols == label_ref[...]
    out_ref[...] = cosine_ref[...] * S - jnp.where(mask, S * M, jnp.float32(0.0))


def _tc_part(cosine, label2d):
    nb = _B_TC // _BLOCK_B
    nc = pl.cdiv(_C, _BLOCK_C)
    return pl.pallas_call(
        _mcp_block,
        grid=(nb, nc),
        in_specs=[
            pl.BlockSpec((_BLOCK_B, _BLOCK_C), lambda i, j: (i, j)),
            pl.BlockSpec((_BLOCK_B, 1), lambda i, j: (i, 0)),
        ],
        out_specs=pl.BlockSpec((_BLOCK_B, _BLOCK_C), lambda i, j: (i, j)),
        out_shape=jax.ShapeDtypeStruct((_B_TC, _C), jnp.float32),
    )(cosine, label2d)


def _sc_body(cos_hbm, fp_hbm, out_hbm, fp_v, vals_v, *bufs_and_sems):
    bufin = bufs_and_sems[:_NBUF]
    bufout = bufs_and_sems[_NBUF:2 * _NBUF]
    in_sems = bufs_and_sems[2 * _NBUF]
    out_sems = bufs_and_sems[2 * _NBUF + 1]
    fix_sem = bufs_and_sems[2 * _NBUF + 2]

    wid = lax.axis_index("s") * 2 + lax.axis_index("c")
    base_row = wid * _ROWS_PW
    in_base = (_B_TC + base_row) * _C
    out_base = base_row * _C

    pltpu.sync_copy(fp_hbm.at[pl.ds(wid * 16, 16)], fp_v)

    def start_in(t, b):
        pltpu.async_copy(
            cos_hbm.at[pl.ds(in_base + t * _CHUNK, _CHUNK)], bufin[b], in_sems.at[b]
        )

    def wait_in(t, b):
        pltpu.make_async_copy(
            cos_hbm.at[pl.ds(in_base + t * _CHUNK, _CHUNK)], bufin[b], in_sems.at[b]
        ).wait()

    def start_out(t, b):
        pltpu.async_copy(
            bufout[b], out_hbm.at[pl.ds(out_base + t * _CHUNK, _CHUNK)], out_sems.at[b]
        )

    def wait_out(t, b):
        pltpu.make_async_copy(
            bufout[b], out_hbm.at[pl.ds(out_base + t * _CHUNK, _CHUNK)], out_sems.at[b]
        ).wait()

    for b in range(_NBUF):
        start_in(b, b)

    def round_body(g, _):
        for b in range(_NBUF):
            t = g * _NBUF + b
            wait_in(t, b)

            @pl.when(g > 0)
            def _():
                wait_out(t - _NBUF, b)

            def vec_body(j, _):
                sl = pl.ds(j * 16, 16)
                bufout[b][sl] = bufin[b][sl] * S
                return 0

            lax.fori_loop(0, _VREGS, vec_body, 0, unroll=25)

            start_out(t, b)

            @pl.when(t + _NBUF < _T)
            def _():
                start_in(t + _NBUF, b)
        return 0

    lax.fori_loop(0, _T // _NBUF, round_body, 0)

    for b in range(_NBUF):
        wait_out(_T - _NBUF + b, b)

    pltpu.async_copy(out_hbm.at[fp_v], vals_v, fix_sem).wait()
    vals_v[...] = vals_v[...] - jnp.float32(S * M)
    pltpu.async_copy(vals_v, out_hbm.at[fp_v], fix_sem).wait()


def _sc_part(cos_flat, fp):
    mesh = plsc.VectorSubcoreMesh(core_axis_name="c", subcore_axis_name="s")
    return pl.kernel(
        _sc_body,
        mesh=mesh,
        out_type=jax.ShapeDtypeStruct((_B_SC * _C,), jnp.float32),
        scratch_types=(
            [pltpu.VMEM((16,), jnp.int32), pltpu.VMEM((16,), jnp.float32)]
            + [pltpu.VMEM((_CHUNK,), jnp.float32) for _ in range(2 * _NBUF)]
            + [
                pltpu.SemaphoreType.DMA((_NBUF,)),
                pltpu.SemaphoreType.DMA((_NBUF,)),
                pltpu.SemaphoreType.DMA,
            ]
        ),
    )(cos_flat, fp)


@jax.jit
def kernel(cosine, label):
    B, C = cosine.shape
    lab32 = label.astype(jnp.int32)
    label2d = lab32.reshape(B, 1)
    cos_flat = cosine.reshape(B * C)
    fp8 = (jnp.arange(_B_SC, dtype=jnp.int32) * C + lab32[_B_TC:]).reshape(_NW, _ROWS_PW)
    fp = jnp.concatenate([fp8, fp8], axis=1).reshape(_NW * 16)

    tc_out = _tc_part(cosine, label2d)
    sc_out = _sc_part(cos_flat, fp)
    return jnp.concatenate([tc_out, sc_out.reshape(_B_SC, C)], axis=0)


# speedup vs baseline: 1.9498x; 1.9498x over previous
import jax
import jax.numpy as jnp
from jax.experimental import pallas as pl

S = 30.0
M = 0.4

_BLOCK_B = 256
_BLOCK_C = 8192


def _mcp_block(cosine_ref, label_ref, out_ref):
    j = pl.program_id(1)
    cols = jax.lax.broadcasted_iota(jnp.int32, cosine_ref.shape, 1) + j * _BLOCK_C
    mask = cols == label_ref[...]
    out_ref[...] = cosine_ref[...] * S - jnp.where(mask, S * M, jnp.float32(0.0))


@jax.jit
def kernel(cosine, label):
    B, C = cosine.shape
    label2d = label.astype(jnp.int32).reshape(B, 1)
    nb = pl.cdiv(B, _BLOCK_B)
    nc = pl.cdiv(C, _BLOCK_C)
    return pl.pallas_call(
        _mcp_block,
        grid=(nb, nc),
        in_specs=[
            pl.BlockSpec((_BLOCK_B, _BLOCK_C), lambda i, j: (i, j)),
            pl.BlockSpec((_BLOCK_B, 1), lambda i, j: (i, 0)),
        ],
        out_specs=pl.BlockSpec((_BLOCK_B, _BLOCK_C), lambda i, j: (i, j)),
        out_shape=jax.ShapeDtypeStruct((B, C), cosine.dtype),
    )(cosine, label2d)


# speedup vs baseline: 1.9589x; 1.0047x over previous
import jax
import jax.numpy as jnp
from jax.experimental import pallas as pl

S = 30.0
M = 0.4

_BLOCK_B = 512
_BLOCK_C = 4096


def _mcp_block(cosine_ref, label_ref, out_ref):
    j = pl.program_id(1)
    cols = jax.lax.broadcasted_iota(jnp.int32, cosine_ref.shape, 1) + j * _BLOCK_C
    mask = cols == label_ref[...]
    out_ref[...] = cosine_ref[...] * S - jnp.where(mask, S * M, jnp.float32(0.0))


@jax.jit
def kernel(cosine, label):
    B, C = cosine.shape
    label2d = label.astype(jnp.int32).reshape(B, 1)
    nb = pl.cdiv(B, _BLOCK_B)
    nc = pl.cdiv(C, _BLOCK_C)
    return pl.pallas_call(
        _mcp_block,
        grid=(nb, nc),
        in_specs=[
            pl.BlockSpec((_BLOCK_B, _BLOCK_C), lambda i, j: (i, j)),
            pl.BlockSpec((_BLOCK_B, 1), lambda i, j: (i, 0)),
        ],
        out_specs=pl.BlockSpec((_BLOCK_B, _BLOCK_C), lambda i, j: (i, j)),
        out_shape=jax.ShapeDtypeStruct((B, C), cosine.dtype),
    )(cosine, label2d)
